# baseline (speedup 1.0000x reference)
import jax
import jax.numpy as jnp
from jax import lax
from jax.experimental import pallas as pl
from jax.experimental.pallas import tpu as pltpu
from jax.experimental.pallas import tpu_sc as plsc

_N = 50000
_D = 128
_C = 40
_E = 800000

_NC = 2
_NS = 16
_NW = _NC * _NS

_CHR = 20
_CH = 24

_CHUNK = 128
_NCHUNK = 196
_EPT = _CHUNK * _NCHUNK
_E_PAD = _NW * _EPT
_TOTCHUNK = _E_PAD // _CHUNK
_ACHUNK = _TOTCHUNK // _NS
_APHASE = 7
_APC = _ACHUNK // _APHASE

_N_PAD = 50048
_ZROWS = _N_PAD // _NS
_ZFULL = _ZROWS // _CHUNK
_ZTAIL = _ZROWS - _ZFULL * _CHUNK

_NBUF = 4
_ABUF = 8
_BLK = 1000


def _h_body(xe_ref, xh_ref, xs_ref, w_ref, h_ref):
    xe = xe_ref[...]
    xh = xh_ref[...]
    xs = xs_ref[...]

    nh = jnp.sqrt(jnp.sum(xh * xh, axis=1, keepdims=True))
    nhc = jnp.clip(nh, 1e-15, 1.0 - 1e-5)
    artanh = 0.5 * jnp.log((1.0 + nhc) / (1.0 - nhc))
    xh_l = artanh * xh / jnp.maximum(nh, 1e-15)

    ns = jnp.sqrt(jnp.sum(xs * xs, axis=1, keepdims=True))
    v1 = ns / (1.0 + jnp.sqrt(1.0 + ns * ns))
    v2 = v1 / (1.0 + jnp.sqrt(1.0 + v1 * v1))
    t2 = v2 * v2
    poly = 1.0 + t2 * (-1.0 / 3 + t2 * (1.0 / 5 + t2 * (-1.0 / 7 + t2 * (
        1.0 / 9 + t2 * (-1.0 / 11 + t2 * (1.0 / 13))))))
    atan_ns = 4.0 * v2 * poly
    xs_l = atan_ns * xs / jnp.maximum(ns, 1e-15)

    h = jnp.dot(xe, w_ref[0:_D, :], preferred_element_type=jnp.float32)
    h += jnp.dot(xh_l, w_ref[_D:2 * _D, :], preferred_element_type=jnp.float32)
    h += jnp.dot(xs_l, w_ref[2 * _D:3 * _D, :], preferred_element_type=jnp.float32)
    h_ref[...] = h


def _compute_h(x_E, x_H, x_S, W):
    grid = (_N // _BLK,)
    xspec = pl.BlockSpec((_BLK, _D), lambda i: (i, 0))
    return pl.pallas_call(
        _h_body,
        grid=grid,
        in_specs=[xspec, xspec, xspec, pl.BlockSpec((3 * _D, _C), lambda i: (0, 0))],
        out_specs=pl.BlockSpec((_BLK, _C), lambda i: (i, 0)),
        out_shape=jax.ShapeDtypeStruct((_N, _C), jnp.float32),
    )(x_E, x_H, x_S, W)


def _rsqrt16(d):
    i = plsc.bitcast(d, jnp.int32)
    y = plsc.bitcast(jnp.int32(0x5F3759DF) - (i >> 1), jnp.float32)
    for _ in range(3):
        y = y * (1.5 - 0.5 * d * y * y)
    return y


def _m1_body(col_hbm, h_hbm, g2_hbm, dinv_hbm,
             cidx2, ones_v, zero_v, dv, hbuf, gbuf, s0, s1, s2, s3, deg_sh):
    cid = lax.axis_index("c")
    sid = lax.axis_index("s")
    ssem = [s0, s1, s2, s3]

    z16 = jnp.zeros((16,), jnp.float32)
    o16 = jnp.ones((16,), jnp.float32)
    for i in range(_CHUNK // 16):
        ones_v[pl.ds(i * 16, 16)] = o16
        zero_v[pl.ds(i * 16, 16)] = z16

    def zloop(t, carry):
        pltpu.sync_copy(zero_v, deg_sh.at[pl.ds(sid * _ZROWS + t * _CHUNK, _CHUNK)])
        return carry
    lax.fori_loop(0, _ZFULL, zloop, 0)
    pltpu.sync_copy(zero_v.at[pl.ds(0, _ZTAIL)],
                    deg_sh.at[pl.ds(sid * _ZROWS + _ZFULL * _CHUNK, _ZTAIL)])
    plsc.subcore_barrier()

    for p in range(_APHASE):
        pltpu.sync_copy(col_hbm.at[pl.ds(sid * _ACHUNK + p * _APC, _APC)], cidx2)

        def grp(gi, carry):
            descs = []
            for b in range(_NBUF):
                j = gi * _NBUF + b
                descs.append(pltpu.async_copy(
                    ones_v, deg_sh.at[cidx2.at[j]], ssem[b], add=True))
            for d in descs:
                d.wait()
            return carry
        lax.fori_loop(0, _APC // _NBUF, grp, 0)
    plsc.subcore_barrier()

    pltpu.sync_copy(deg_sh.at[pl.ds(sid * _ZROWS, _ZROWS)], dv.at[pl.ds(0, _ZROWS)])

    def newt(k, carry):
        d = dv[pl.ds(k * 16, 16)] + 1.0
        dv[pl.ds(k * 16, 16)] = _rsqrt16(d)
        return carry
    lax.fori_loop(0, (_ZROWS + 15) // 16, newt, 0)
    pltpu.sync_copy(dv.at[pl.ds(0, _ZROWS)], deg_sh.at[pl.ds(sid * _ZROWS, _ZROWS)])
    pltpu.sync_copy(dv.at[pl.ds(0, _ZROWS)],
                    dinv_hbm.at[cid, pl.ds(sid * _ZROWS, _ZROWS)])

    choff = cid * 16

    def scale_rows(nrows, rowbase):
        pltpu.sync_copy(h_hbm.at[pl.ds(rowbase, nrows)], hbuf.at[pl.ds(0, nrows)])

        def srow(r, carry):
            dsp = plsc.load_gather(dv, [jnp.zeros((16,), jnp.int32) + (carry + r)])
            gbuf[r, pl.ds(0, 16)] = hbuf[r, pl.ds(choff, 16)] * dsp
            gbuf[r, pl.ds(8, 16)] = hbuf[r, pl.ds(choff + 8, 16)] * dsp
            return carry
        lax.fori_loop(0, nrows, srow, rowbase - sid * _ZROWS)
        pltpu.sync_copy(gbuf.at[pl.ds(0, nrows)],
                        g2_hbm.at[cid, pl.ds(rowbase, nrows)])

    def cloop(t, carry):
        scale_rows(_CHUNK, sid * _ZROWS + t * _CHUNK)
        return carry
    lax.fori_loop(0, _ZFULL, cloop, 0)
    @pl.when(sid < _NS - 1)
    def _():
        scale_rows(_ZTAIL, sid * _ZROWS + _ZFULL * _CHUNK)

    @pl.when(sid == _NS - 1)
    def _():
        scale_rows(8, (_NS - 1) * _ZROWS + _ZFULL * _CHUNK)


def _compute_m1(col2, h):
    mesh = plsc.VectorSubcoreMesh(core_axis_name="c", subcore_axis_name="s")
    f = pl.kernel(
        _m1_body,
        out_type=(jax.ShapeDtypeStruct((_NC, _N_PAD, _CH), jnp.float32),
                  jax.ShapeDtypeStruct((_NC, _N_PAD), jnp.float32)),
        mesh=mesh,
        scratch_types=[
            pltpu.VMEM((_APC, _CHUNK), jnp.int32),
            pltpu.VMEM((_CHUNK,), jnp.float32),
            pltpu.VMEM((_CHUNK,), jnp.float32),
            pltpu.VMEM((_ZROWS + 8, ), jnp.float32),
            pltpu.VMEM((_CHUNK, _C), jnp.float32),
            pltpu.VMEM((_CHUNK, _CH), jnp.float32),
            *([pltpu.SemaphoreType.DMA] * _NBUF),
            pltpu.VMEM_SHARED((_N_PAD,), jnp.float32),
        ],
        compiler_params=pltpu.CompilerParams(use_tc_tiling_on_sc=False, needs_layout_passes=False),
    )
    return f(col2, h)


def _agg_body(row_hbm, col_hbm, g2_hbm, dinv_hbm, s_hbm, ridx2, cidx2,
              r0, r1, r2, r3, r4, r5, r6, r7,
              g0, g1, g2, g3, g4, g5, g6, g7, ddbuf, acc_sh):
    cid = lax.axis_index("c")
    sid = lax.axis_index("s")
    rows = [r0, r1, r2, r3, r4, r5, r6, r7]
    gsem = [g0, g1, g2, g3, g4, g5, g6, g7]
    gsrc = g2_hbm.at[cid]

    z16 = jnp.zeros((16,), jnp.float32)

    def zrow(i, carry):
        rows[0][i, pl.ds(0, 16)] = z16
        rows[0][i, pl.ds(8, 16)] = z16
        return carry
    lax.fori_loop(0, _CHUNK, zrow, 0)

    def zacc(t, carry):
        pltpu.sync_copy(rows[0], acc_sh.at[pl.ds(sid * _ZROWS + t * _CHUNK, _CHUNK)])
        return carry
    lax.fori_loop(0, _ZFULL, zacc, 0)
    pltpu.sync_copy(rows[0].at[pl.ds(0, _ZTAIL)],
                    acc_sh.at[pl.ds(sid * _ZROWS + _ZFULL * _CHUNK, _ZTAIL)])
    plsc.subcore_barrier()

    for p in range(_APHASE):
        cbase = sid * _ACHUNK + p * _APC
        pltpu.sync_copy(row_hbm.at[pl.ds(cbase, _APC)], ridx2)
        pltpu.sync_copy(col_hbm.at[pl.ds(cbase, _APC)], cidx2)

        def grp(gi, carry):
            descs = []
            for b in range(_ABUF):
                j = gi * _ABUF + b
                descs.append(pltpu.async_copy(
                    gsrc.at[ridx2.at[j]], rows[b], gsem[b]))
            for b in range(_ABUF):
                j = gi * _ABUF + b
                descs[b].wait()
                pltpu.sync_copy(rows[b], acc_sh.at[cidx2.at[j]], add=True)
            return carry
        lax.fori_loop(0, _APC // _ABUF, grp, 0)

    plsc.subcore_barrier()

    def finale(t, carry):
        rowbase = sid * _ZROWS + t * _CHUNK
        pltpu.sync_copy(acc_sh.at[pl.ds(rowbase, _CHUNK)], rows[0])
        pltpu.sync_copy(gsrc.at[pl.ds(rowbase, _CHUNK)], rows[1])
        pltpu.sync_copy(dinv_hbm.at[cid, pl.ds(rowbase, _CHUNK)], ddbuf)

        def frow(r, c):
            dsp = plsc.load_gather(ddbuf, [jnp.zeros((16,), jnp.int32) + r])
            v0 = (rows[0][r, pl.ds(0, 16)] + rows[1][r, pl.ds(0, 16)]) * dsp
            v1 = (rows[0][r, pl.ds(8, 16)] + rows[1][r, pl.ds(8, 16)]) * dsp
            rows[0][r, pl.ds(0, 16)] = v0
            rows[0][r, pl.ds(8, 16)] = v1
            return c
        lax.fori_loop(0, _CHUNK, frow, 0)
        pltpu.sync_copy(rows[0], s_hbm.at[cid, pl.ds(rowbase, _CHUNK)])
        return carry
    lax.fori_loop(0, _ZFULL, finale, 0)
    rowbase = sid * _ZROWS + _ZFULL * _CHUNK
    pltpu.sync_copy(acc_sh.at[pl.ds(rowbase, _ZTAIL)], rows[0].at[pl.ds(0, _ZTAIL)])
    pltpu.sync_copy(gsrc.at[pl.ds(rowbase, _ZTAIL)], rows[1].at[pl.ds(0, _ZTAIL)])
    pltpu.sync_copy(dinv_hbm.at[cid, pl.ds(rowbase, _ZTAIL)], ddbuf.at[pl.ds(0, _ZTAIL)])

    def frow2(r, c):
        dsp = plsc.load_gather(ddbuf, [jnp.zeros((16,), jnp.int32) + r])
        v0 = (rows[0][r, pl.ds(0, 16)] + rows[1][r, pl.ds(0, 16)]) * dsp
        v1 = (rows[0][r, pl.ds(8, 16)] + rows[1][r, pl.ds(8, 16)]) * dsp
        rows[0][r, pl.ds(0, 16)] = v0
        rows[0][r, pl.ds(8, 16)] = v1
        return c
    lax.fori_loop(0, _ZTAIL, frow2, 0)
    pltpu.sync_copy(rows[0].at[pl.ds(0, _ZTAIL)], s_hbm.at[cid, pl.ds(rowbase, _ZTAIL)])


def _compute_s(row2, col2, g2, dinv2):
    mesh = plsc.VectorSubcoreMesh(core_axis_name="c", subcore_axis_name="s")
    f = pl.kernel(
        _agg_body,
        out_type=jax.ShapeDtypeStruct((_NC, _N_PAD, _CH), jnp.float32),
        mesh=mesh,
        scratch_types=[
            pltpu.VMEM((_APC, _CHUNK), jnp.int32),
            pltpu.VMEM((_APC, _CHUNK), jnp.int32),
            *([pltpu.VMEM((_CHUNK, _CH), jnp.float32)] * _ABUF),
            *([pltpu.SemaphoreType.DMA] * _ABUF),
            pltpu.VMEM((_CHUNK,), jnp.float32),
            pltpu.VMEM_SHARED((_N_PAD, _CH), jnp.float32),
        ],
        compiler_params=pltpu.CompilerParams(use_tc_tiling_on_sc=False, needs_layout_passes=False),
    )
    return f(row2, col2, g2, dinv2)


def kernel(x_E, x_H, x_S, W, edge_index):
    npad = _E_PAD - _E
    row2 = jnp.concatenate(
        [edge_index[0], jnp.zeros((npad,), jnp.int32)]).reshape(
            _TOTCHUNK, _CHUNK)
    col2 = jnp.concatenate(
        [edge_index[1], jnp.full((npad,), _N, jnp.int32)]).reshape(
            _TOTCHUNK, _CHUNK)

    h = _compute_h(x_E, x_H, x_S, W)
    g2, dinv2 = _compute_m1(col2, h)
    sout = _compute_s(row2, col2, g2, dinv2)
    return jnp.concatenate([sout[0, :_N, 0:_CHR], sout[1, :_N, 4:4 + _CHR]],
                           axis=1)


# speedup vs baseline: 1.1689x; 1.1689x over previous
import jax
import jax.numpy as jnp
from jax import lax
from jax.experimental import pallas as pl
from jax.experimental.pallas import tpu as pltpu
from jax.experimental.pallas import tpu_sc as plsc

_N = 50000
_D = 128
_C = 40
_E = 800000

_NC = 2
_NS = 16
_NW = _NC * _NS

_CHR = _C // _NC
_CH = 24

_CHUNK = 128
_NCHUNK = 196
_EPT = _CHUNK * _NCHUNK
_E_PAD = _NW * _EPT
_TOTCHUNK = _E_PAD // _CHUNK
_ACHUNK = _TOTCHUNK // _NS
_APHASE = 7
_APC = _ACHUNK // _APHASE

_N_PAD = 50048
_ZROWS = _N_PAD // _NS
_ZFULL = _ZROWS // _CHUNK
_ZTAIL = _ZROWS - _ZFULL * _CHUNK

_NBUF = 4
_BLK = 1000


def _h_body(xe_ref, xh_ref, xs_ref, w_ref, deg_ref, g2_ref):
    xe = xe_ref[...]
    xh = xh_ref[...]
    xs = xs_ref[...]

    nh = jnp.sqrt(jnp.sum(xh * xh, axis=1, keepdims=True))
    nhc = jnp.clip(nh, 1e-15, 1.0 - 1e-5)
    artanh = 0.5 * jnp.log((1.0 + nhc) / (1.0 - nhc))
    xh_l = artanh * xh / jnp.maximum(nh, 1e-15)

    ns = jnp.sqrt(jnp.sum(xs * xs, axis=1, keepdims=True))
    v1 = ns / (1.0 + jnp.sqrt(1.0 + ns * ns))
    v2 = v1 / (1.0 + jnp.sqrt(1.0 + v1 * v1))
    t2 = v2 * v2
    poly = 1.0 + t2 * (-1.0 / 3 + t2 * (1.0 / 5 + t2 * (-1.0 / 7 + t2 * (
        1.0 / 9 + t2 * (-1.0 / 11 + t2 * (1.0 / 13))))))
    atan_ns = 4.0 * v2 * poly
    xs_l = atan_ns * xs / jnp.maximum(ns, 1e-15)

    h = jnp.dot(xe, w_ref[0:_D, :], preferred_element_type=jnp.float32)
    h += jnp.dot(xh_l, w_ref[_D:2 * _D, :], preferred_element_type=jnp.float32)
    h += jnp.dot(xs_l, w_ref[2 * _D:3 * _D, :], preferred_element_type=jnp.float32)

    deg = deg_ref[:, 0] + deg_ref[:, 1] + 1.0
    dinv = lax.rsqrt(deg)
    g = h * dinv[:, None]
    g2_ref[0] = g[:, 0:_CH]
    g2_ref[1] = g[:, _C - _CH:_C]


def _compute_g2(x_E, x_H, x_S, W, degp_t):
    grid = (_N // _BLK,)
    xspec = pl.BlockSpec((_BLK, _D), lambda i: (i, 0))
    return pl.pallas_call(
        _h_body,
        grid=grid,
        in_specs=[xspec, xspec, xspec, pl.BlockSpec((3 * _D, _C), lambda i: (0, 0)),
                  pl.BlockSpec((_BLK, _NC), lambda i: (i, 0))],
        out_specs=pl.BlockSpec((_NC, _BLK, _CH), lambda i: (0, i, 0)),
        out_shape=jax.ShapeDtypeStruct((_NC, _N, _CH), jnp.float32),
    )(x_E, x_H, x_S, W, degp_t)


def _deg_body(col_hbm, deg_hbm, cidx2, ones_v, zero_v, s0, s1, s2, s3, deg_sh):
    cid = lax.axis_index("c")
    sid = lax.axis_index("s")
    wid = cid * _NS + sid
    ssem = [s0, s1, s2, s3]

    z16 = jnp.zeros((16,), jnp.float32)
    o16 = jnp.ones((16,), jnp.float32)
    for i in range(_CHUNK // 16):
        ones_v[pl.ds(i * 16, 16)] = o16
        zero_v[pl.ds(i * 16, 16)] = z16

    pltpu.sync_copy(col_hbm.at[pl.ds(wid * _NCHUNK, _NCHUNK)], cidx2)

    def zloop(t, carry):
        pltpu.sync_copy(zero_v, deg_sh.at[pl.ds(sid * _ZROWS + t * _CHUNK, _CHUNK)])
        return carry
    lax.fori_loop(0, _ZFULL, zloop, 0)
    pltpu.sync_copy(zero_v.at[pl.ds(0, _ZTAIL)],
                    deg_sh.at[pl.ds(sid * _ZROWS + _ZFULL * _CHUNK, _ZTAIL)])
    plsc.subcore_barrier()

    ngrp = _NCHUNK // _NBUF
    def grp(gi, carry):
        descs = []
        for b in range(_NBUF):
            j = gi * _NBUF + b
            descs.append(pltpu.async_copy(
                ones_v, deg_sh.at[cidx2.at[j]], ssem[b], add=True))
        for d in descs:
            d.wait()
        return carry
    lax.fori_loop(0, ngrp, grp, 0)
    plsc.subcore_barrier()

    pltpu.sync_copy(deg_sh.at[pl.ds(sid * _ZROWS, _ZROWS)],
                    deg_hbm.at[cid, pl.ds(sid * _ZROWS, _ZROWS)])


def _compute_deg(col2):
    mesh = plsc.VectorSubcoreMesh(core_axis_name="c", subcore_axis_name="s")
    f = pl.kernel(
        _deg_body,
        out_type=jax.ShapeDtypeStruct((_NC, _N_PAD), jnp.float32),
        mesh=mesh,
        scratch_types=[
            pltpu.VMEM((_NCHUNK, _CHUNK), jnp.int32),
            pltpu.VMEM((_CHUNK,), jnp.float32),
            pltpu.VMEM((_CHUNK,), jnp.float32),
            pltpu.SemaphoreType.DMA,
            pltpu.SemaphoreType.DMA,
            pltpu.SemaphoreType.DMA,
            pltpu.SemaphoreType.DMA,
            pltpu.VMEM_SHARED((_N_PAD,), jnp.float32),
        ],
        compiler_params=pltpu.CompilerParams(use_tc_tiling_on_sc=False),
    )
    return f(col2)


_ABUF = 8


def _agg_body(row_hbm, col_hbm, g2_hbm, s_hbm, ridx2, cidx2,
              r0, r1, r2, r3, r4, r5, r6, r7,
              g0, g1, g2, g3, g4, g5, g6, g7, acc_sh):
    cid = lax.axis_index("c")
    sid = lax.axis_index("s")
    rows = [r0, r1, r2, r3, r4, r5, r6, r7]
    gsem = [g0, g1, g2, g3, g4, g5, g6, g7]
    gsrc = g2_hbm.at[cid]

    z16 = jnp.zeros((16,), jnp.float32)

    def zrow(i, carry):
        rows[0][i, pl.ds(0, 16)] = z16
        rows[0][i, pl.ds(8, 16)] = z16
        return carry
    lax.fori_loop(0, _CHUNK, zrow, 0)

    def zacc(t, carry):
        pltpu.sync_copy(rows[0], acc_sh.at[pl.ds(sid * _ZROWS + t * _CHUNK, _CHUNK)])
        return carry
    lax.fori_loop(0, _ZFULL, zacc, 0)
    pltpu.sync_copy(rows[0].at[pl.ds(0, _ZTAIL)],
                    acc_sh.at[pl.ds(sid * _ZROWS + _ZFULL * _CHUNK, _ZTAIL)])
    plsc.subcore_barrier()

    for p in range(_APHASE):
        cbase = sid * _ACHUNK + p * _APC
        pltpu.sync_copy(row_hbm.at[pl.ds(cbase, _APC)], ridx2)
        pltpu.sync_copy(col_hbm.at[pl.ds(cbase, _APC)], cidx2)

        ngrp = _APC // _ABUF
        def grp(gi, carry):
            descs = []
            for b in range(_ABUF):
                j = gi * _ABUF + b
                descs.append(pltpu.async_copy(
                    gsrc.at[ridx2.at[j]], rows[b], gsem[b]))
            for b in range(_ABUF):
                j = gi * _ABUF + b
                descs[b].wait()
                pltpu.sync_copy(rows[b], acc_sh.at[cidx2.at[j]], add=True)
            return carry
        lax.fori_loop(0, ngrp, grp, 0)

    plsc.subcore_barrier()
    pltpu.sync_copy(acc_sh.at[pl.ds(sid * _ZROWS, _ZROWS)],
                    s_hbm.at[cid, pl.ds(sid * _ZROWS, _ZROWS)])


def _compute_s(row2, col2, g2):
    mesh = plsc.VectorSubcoreMesh(core_axis_name="c", subcore_axis_name="s")
    f = pl.kernel(
        _agg_body,
        out_type=jax.ShapeDtypeStruct((_NC, _N_PAD, _CH), jnp.float32),
        mesh=mesh,
        scratch_types=[
            pltpu.VMEM((_APC, _CHUNK), jnp.int32),
            pltpu.VMEM((_APC, _CHUNK), jnp.int32),
            *([pltpu.VMEM((_CHUNK, _CH), jnp.float32)] * _ABUF),
            *([pltpu.SemaphoreType.DMA] * _ABUF),
            pltpu.VMEM_SHARED((_N_PAD, _CH), jnp.float32),
        ],
        compiler_params=pltpu.CompilerParams(use_tc_tiling_on_sc=False),
    )
    return f(row2, col2, g2)


def _out_body(s_ref, g_ref, deg_ref, o_ref):
    deg = deg_ref[:, 0] + deg_ref[:, 1] + 1.0
    dinv = lax.rsqrt(deg)
    tot = jnp.concatenate([(s_ref[0] + g_ref[0])[:, 0:_CHR],
                           (s_ref[1] + g_ref[1])[:, _CH - _CHR:_CH]], axis=1)
    o_ref[...] = tot * dinv[:, None]


def _compute_out(s, g2, degp_t):
    grid = (_N // _BLK,)
    return pl.pallas_call(
        _out_body,
        grid=grid,
        in_specs=[pl.BlockSpec((_NC, _BLK, _CH), lambda i: (0, i, 0)),
                  pl.BlockSpec((_NC, _BLK, _CH), lambda i: (0, i, 0)),
                  pl.BlockSpec((_BLK, _NC), lambda i: (i, 0))],
        out_specs=pl.BlockSpec((_BLK, _C), lambda i: (i, 0)),
        out_shape=jax.ShapeDtypeStruct((_N, _C), jnp.float32),
    )(s, g2, degp_t)


def kernel(x_E, x_H, x_S, W, edge_index):
    npad = _E_PAD - _E
    row2 = jnp.concatenate(
        [edge_index[0], jnp.zeros((npad,), jnp.int32)]).reshape(
            _TOTCHUNK, _CHUNK)
    col2 = jnp.concatenate(
        [edge_index[1], jnp.full((npad,), _N, jnp.int32)]).reshape(
            _TOTCHUNK, _CHUNK)

    degp = _compute_deg(col2)
    degp_t = degp.T
    g2 = _compute_g2(x_E, x_H, x_S, W, degp_t)
    s = _compute_s(row2, col2, g2)
    return _compute_out(s, g2, degp_t)
